# baseline (speedup 1.0000x reference)
import functools

import jax
import jax.numpy as jnp
from jax import lax
from jax.experimental import pallas as pl
from jax.experimental.pallas import tpu as pltpu
from jax.experimental.pallas import tpu_sc as plsc

B, S, D = 2, 2048, 1024
H = 2048
E = 8
TOPK = 2
N = B * S
NA = N * TOPK
T = 256
NB = NA // T + E
NBPAD = 48
G = NB * T
GW = 32
CW = 16

@functools.cache
def _sc_mesh():
    return plsc.VectorSubcoreMesh(core_axis_name="c", subcore_axis_name="s")


_SC_PARAMS = pltpu.CompilerParams(needs_layout_passes=False)


def _gelu(x):
    return 0.5 * x * (1.0 + lax.erf(x * (2.0 ** -0.5)))


def _router_body(x_ref, wg_ref, b_ref, topi_ref, topw_ref, xb_ref):
    x = x_ref[...]
    xb_ref[...] = x.astype(jnp.bfloat16)
    logits = lax.dot_general(x, wg_ref[...], (((1,), (1,)), ((), ())),
                             preferred_element_type=jnp.float32)
    logits = logits + b_ref[...]
    iota = lax.broadcasted_iota(jnp.int32, logits.shape, 1)
    v0 = jnp.max(logits, axis=1, keepdims=True)
    i0 = jnp.min(jnp.where(logits == v0, iota, E), axis=1, keepdims=True)
    l2 = jnp.where(iota == i0, -jnp.inf, logits)
    v1 = jnp.max(l2, axis=1, keepdims=True)
    i1 = jnp.min(jnp.where(l2 == v1, iota, E), axis=1, keepdims=True)
    s1 = jnp.exp(v1 - v0)
    w0 = 1.0 / (1.0 + s1)
    topi_ref[...] = jnp.concatenate([i0, i1], axis=1)
    topw_ref[...] = jnp.concatenate([w0, 1.0 - w0], axis=1)


def _router(xf, Wg, bias_buf):
    blk = 1024
    return pl.pallas_call(
        _router_body,
        grid=(N // blk,),
        in_specs=[
            pl.BlockSpec((blk, D), lambda i: (i, 0)),
            pl.BlockSpec((E, D), lambda i: (0, 0)),
            pl.BlockSpec((1, E), lambda i: (0, 0)),
        ],
        out_specs=[
            pl.BlockSpec((blk, TOPK), lambda i: (i, 0)),
            pl.BlockSpec((blk, TOPK), lambda i: (i, 0)),
            pl.BlockSpec((blk, D), lambda i: (i, 0)),
        ],
        out_shape=[
            jax.ShapeDtypeStruct((N, TOPK), jnp.int32),
            jax.ShapeDtypeStruct((N, TOPK), jnp.float32),
            jax.ShapeDtypeStruct((N, D), jnp.bfloat16),
        ],
    )(xf, Wg, bias_buf.reshape(1, E))


def _shared_body(x_ref, w1_ref, w2_ref, b_ref, o_ref):
    h = lax.dot_general(x_ref[...], w1_ref[...], (((1,), (1,)), ((), ())),
                        preferred_element_type=jnp.float32)
    h = _gelu(h).astype(jnp.bfloat16)
    o = lax.dot_general(h, w2_ref[...], (((1,), (1,)), ((), ())),
                        preferred_element_type=jnp.float32)
    o_ref[...] = o + b_ref[...]


def _shared_mlp(xb, Ws1, Ws2, bs2):
    blk = 512
    return pl.pallas_call(
        _shared_body,
        grid=(N // blk,),
        in_specs=[
            pl.BlockSpec((blk, D), lambda i: (i, 0)),
            pl.BlockSpec((H, D), lambda i: (0, 0)),
            pl.BlockSpec((D, H), lambda i: (0, 0)),
            pl.BlockSpec((1, D), lambda i: (0, 0)),
        ],
        out_specs=pl.BlockSpec((blk, D), lambda i: (i, 0)),
        out_shape=jax.ShapeDtypeStruct((N, D), jnp.float32),
    )(xb, Ws1.astype(jnp.bfloat16), Ws2.astype(jnp.bfloat16),
      bs2.reshape(1, D))


def _meta_body(topi_hbm, st_hbm, inv_hbm, be_hbm, topi_v, st_v, inv_v, be_v):
    c = lax.axis_index("c")
    s = lax.axis_index("s")

    @pl.when(jnp.logical_and(c == 0, s == 0))
    def _():
        pltpu.sync_copy(topi_hbm, topi_v)

        zi = jnp.zeros((16,), jnp.int32)

        @pl.loop(0, G, step=16)
        def _(i):
            st_v[pl.ds(i, 16)] = zi

        def cbody(i, cnts):
            chunk = topi_v[pl.ds(i * 16, 16)]
            return tuple(cnts[e] + jnp.sum((chunk == e).astype(jnp.int32))
                         for e in range(E))

        cnt = lax.fori_loop(0, NA // 16, cbody,
                            tuple(jnp.int32(0) for _ in range(E)))

        pstart = []
        acc = jnp.int32(0)
        for e in range(E):
            pstart.append(acc)
            acc = acc + ((cnt[e] + (T - 1)) // T) * T
        bstart = [p // T for p in pstart]

        @pl.loop(0, NBPAD, step=16)
        def _(j):
            bvec = lax.iota(jnp.int32, 16) + j
            accv = jnp.full((16,), -1, jnp.int32)
            for e in range(E):
                accv = accv + (bvec >= bstart[e]).astype(jnp.int32)
            be_v[pl.ds(j, 16)] = jnp.minimum(accv, E - 1)

        def pbody(i, off):
            base = i * 16
            chunk = topi_v[pl.ds(base, 16)]
            avec = lax.iota(jnp.int32, 16) + base
            tok = avec >> 1
            inv_idx = (avec & 1) * N + (avec >> 1)
            inv_acc = jnp.zeros((16,), jnp.int32)
            newoff = []
            for e in range(E):
                m = chunk == e
                mi = m.astype(jnp.int32)
                r = jnp.cumsum(mi) - 1
                dest = off[e] + r
                plsc.store_scatter(st_v, [dest], tok, mask=m)
                inv_acc = jnp.where(m, dest, inv_acc)
                newoff.append(off[e] + jnp.sum(mi))
            plsc.store_scatter(inv_v, [inv_idx], inv_acc)
            return tuple(newoff)

        lax.fori_loop(0, NA // 16, pbody, tuple(pstart))

        pltpu.sync_copy(st_v, st_hbm)
        pltpu.sync_copy(inv_v, inv_hbm)
        pltpu.sync_copy(be_v, be_hbm)


def _meta(topi_flat):
    return pl.kernel(
        _meta_body,
        out_type=(
            jax.ShapeDtypeStruct((G,), jnp.int32),
            jax.ShapeDtypeStruct((NA,), jnp.int32),
            jax.ShapeDtypeStruct((NBPAD,), jnp.int32),
        ),
        mesh=_sc_mesh(),
        compiler_params=_SC_PARAMS,
        scratch_types=[
            pltpu.VMEM((NA,), jnp.int32),
            pltpu.VMEM((G,), jnp.int32),
            pltpu.VMEM((NA,), jnp.int32),
            pltpu.VMEM((NBPAD,), jnp.int32),
        ],
    )(topi_flat)


_RPW = G // 32
_DW = D // 2
_NBUF = 4
_NWIN = _RPW // GW


def _gather_body(x_hbm, st_hbm, xs_hbm, idx_v, *bufs_sems):
    bufs = bufs_sems[:_NBUF]
    sems = bufs_sems[_NBUF:]
    c = lax.axis_index("c")
    s = lax.axis_index("s")
    wid = s * 2 + c
    base = wid * _RPW
    pltpu.sync_copy(st_hbm.at[pl.ds(base, _RPW)], idx_v)

    def fire(w, b):
        pltpu.async_copy(x_hbm.at[idx_v.at[pl.ds(w * GW, GW)]],
                         bufs[b], sems[b])

    def drain(w, b):
        pltpu.make_async_copy(x_hbm.at[idx_v.at[pl.ds(w * GW, GW)]],
                              bufs[b], sems[b]).wait()

    for k in range(min(_NBUF, _NWIN)):
        fire(k, k)
    for w in range(_NWIN):
        b = w % _NBUF
        drain(w, b)
        pltpu.sync_copy(bufs[b], xs_hbm.at[pl.ds(base + w * GW, GW)])
        if w + _NBUF < _NWIN:
            fire(w + _NBUF, b)


def _gather(xb32, st):
    return pl.kernel(
        _gather_body,
        out_type=jax.ShapeDtypeStruct((G, _DW), jnp.int32),
        mesh=_sc_mesh(),
        compiler_params=_SC_PARAMS,
        scratch_types=[pltpu.VMEM((_RPW,), jnp.int32)]
        + [pltpu.VMEM((GW, _DW), jnp.int32) for _ in range(_NBUF)]
        + [pltpu.SemaphoreType.DMA for _ in range(_NBUF)],
    )(xb32, st)


def _grouped_body(be_ref, xs_ref, w1_ref, w2_ref, b2_ref, o_ref):
    h = lax.dot_general(xs_ref[...], w1_ref[0], (((1,), (1,)), ((), ())),
                        preferred_element_type=jnp.float32)
    h = _gelu(h).astype(jnp.bfloat16)
    o = lax.dot_general(h, w2_ref[0], (((1,), (1,)), ((), ())),
                        preferred_element_type=jnp.float32)
    o_ref[...] = o + b2_ref[0]


def _grouped(be, xs, W1, W2, b2):
    grid_spec = pltpu.PrefetchScalarGridSpec(
        num_scalar_prefetch=1,
        grid=(NB,),
        in_specs=[
            pl.BlockSpec((T, D), lambda i, be: (i, 0)),
            pl.BlockSpec((1, H, D), lambda i, be: (be[i], 0, 0)),
            pl.BlockSpec((1, D, H), lambda i, be: (be[i], 0, 0)),
            pl.BlockSpec((1, 1, D), lambda i, be: (be[i], 0, 0)),
        ],
        out_specs=pl.BlockSpec((T, D), lambda i, be: (i, 0)),
    )
    return pl.pallas_call(
        _grouped_body,
        grid_spec=grid_spec,
        out_shape=jax.ShapeDtypeStruct((G, D), jnp.float32),
    )(be, xs, W1, W2, b2.reshape(E, 1, D))


_TPW = N // 32


def _combine_body(sh_hbm, os_hbm, inv_hbm, w_hbm, out_hbm,
                  i0_v, i1_v, w0_v, w1_v, sh_v, t0, t1, o_v,
                  sem0, sem1, sem2):
    c = lax.axis_index("c")
    s = lax.axis_index("s")
    wid = s * 2 + c
    base = wid * _TPW
    pltpu.sync_copy(inv_hbm.at[0, pl.ds(base, _TPW)], i0_v)
    pltpu.sync_copy(inv_hbm.at[1, pl.ds(base, _TPW)], i1_v)
    pltpu.sync_copy(w_hbm.at[0, pl.ds(base, _TPW)], w0_v)
    pltpu.sync_copy(w_hbm.at[1, pl.ds(base, _TPW)], w1_v)

    @pl.loop(0, _TPW, step=CW)
    def _(j):
        pltpu.async_copy(os_hbm.at[i0_v.at[pl.ds(j, CW)]], t0, sem0)
        pltpu.async_copy(os_hbm.at[i1_v.at[pl.ds(j, CW)]], t1, sem1)
        pltpu.async_copy(sh_hbm.at[pl.ds(base + j, CW)], sh_v, sem2)
        pltpu.make_async_copy(os_hbm.at[i0_v.at[pl.ds(j, CW)]], t0, sem0).wait()
        pltpu.make_async_copy(os_hbm.at[i1_v.at[pl.ds(j, CW)]], t1, sem1).wait()
        pltpu.make_async_copy(sh_hbm.at[pl.ds(base + j, CW)], sh_v, sem2).wait()

        @pl.loop(0, CW)
        def _(r):
            ri = jnp.full((16,), j + r, jnp.int32)
            w0 = plsc.load_gather(w0_v, [ri])
            w1 = plsc.load_gather(w1_v, [ri])

            @pl.loop(0, D, step=16)
            def _(cc):
                o_v[r, pl.ds(cc, 16)] = (sh_v[r, pl.ds(cc, 16)]
                                         + w0 * t0[r, pl.ds(cc, 16)]
                                         + w1 * t1[r, pl.ds(cc, 16)])

        pltpu.sync_copy(o_v, out_hbm.at[pl.ds(base + j, CW)])


def _combine(shared, os, inv, w01):
    return pl.kernel(
        _combine_body,
        out_type=jax.ShapeDtypeStruct((N, D), jnp.float32),
        mesh=_sc_mesh(),
        compiler_params=_SC_PARAMS,
        scratch_types=[
            pltpu.VMEM((_TPW,), jnp.int32),
            pltpu.VMEM((_TPW,), jnp.int32),
            pltpu.VMEM((_TPW,), jnp.float32),
            pltpu.VMEM((_TPW,), jnp.float32),
            pltpu.VMEM((CW, D), jnp.float32),
            pltpu.VMEM((CW, D), jnp.float32),
            pltpu.VMEM((CW, D), jnp.float32),
            pltpu.VMEM((CW, D), jnp.float32),
            pltpu.SemaphoreType.DMA,
            pltpu.SemaphoreType.DMA,
            pltpu.SemaphoreType.DMA,
        ],
    )(shared, os, inv.reshape(2, N), w01)


def kernel(x, Wg, bias_buf, Ws1, Ws2, bs2, W1, W2, b2):
    xf = x.reshape(N, D)
    topi, topw, xb = _router(xf, Wg, bias_buf)
    shared = _shared_mlp(xb, Ws1, Ws2, bs2)
    st, inv, be = _meta(topi.reshape(NA))
    xb32 = lax.bitcast_convert_type(xb.reshape(N, _DW, 2), jnp.int32)
    xs32 = _gather(xb32, st)
    xsb = lax.bitcast_convert_type(xs32, jnp.bfloat16).reshape(G, D)
    os = _grouped(be, xsb, W1.astype(jnp.bfloat16), W2.astype(jnp.bfloat16),
                  b2)
    w01 = topw.T.reshape(2, N)
    outf = _combine(shared, os, inv, w01)
    return outf.reshape(B, S, D)


# speedup vs baseline: 1.6335x; 1.6335x over previous
import functools

import jax
import jax.numpy as jnp
from jax import lax
from jax.experimental import pallas as pl
from jax.experimental.pallas import tpu as pltpu
from jax.experimental.pallas import tpu_sc as plsc

B, S, D = 2, 2048, 1024
H = 2048
E = 8
TOPK = 2
N = B * S
NA = N * TOPK
T = 256
NB = NA // T + E
NBPAD = 48
G = NB * T
GW = 32
CW = 16

@functools.cache
def _sc_mesh():
    return plsc.VectorSubcoreMesh(core_axis_name="c", subcore_axis_name="s")


_SC_PARAMS = pltpu.CompilerParams(needs_layout_passes=False)


def _gelu(x):
    return 0.5 * x * (1.0 + lax.erf(x * (2.0 ** -0.5)))


_DW = D // 2


def _pack_bf16(x):
    rlo = x[:, :_DW].astype(jnp.bfloat16).astype(jnp.float32)
    rhi = x[:, _DW:].astype(jnp.bfloat16).astype(jnp.float32)
    blo = lax.bitcast_convert_type(rlo, jnp.uint32) >> 16
    bhi = lax.bitcast_convert_type(rhi, jnp.uint32) & jnp.uint32(0xFFFF0000)
    return lax.bitcast_convert_type(blo | bhi, jnp.int32)


def _unpack_bf16(w32):
    u = lax.bitcast_convert_type(w32, jnp.uint32)
    lo = lax.bitcast_convert_type(u << 16, jnp.float32).astype(jnp.bfloat16)
    hi = lax.bitcast_convert_type(u & jnp.uint32(0xFFFF0000),
                                  jnp.float32).astype(jnp.bfloat16)
    return jnp.concatenate([lo, hi], axis=1)


def _router_body(x_ref, wg_ref, b_ref, topi_ref, topw_ref, xb_ref):
    x = x_ref[...]
    xb_ref[...] = _pack_bf16(x)
    logits = lax.dot_general(x, wg_ref[...], (((1,), (1,)), ((), ())),
                             preferred_element_type=jnp.float32)
    logits = logits + b_ref[...]
    iota = lax.broadcasted_iota(jnp.int32, logits.shape, 1)
    v0 = jnp.max(logits, axis=1, keepdims=True)
    i0 = jnp.min(jnp.where(logits == v0, iota, E), axis=1, keepdims=True)
    l2 = jnp.where(iota == i0, -jnp.inf, logits)
    v1 = jnp.max(l2, axis=1, keepdims=True)
    i1 = jnp.min(jnp.where(l2 == v1, iota, E), axis=1, keepdims=True)
    s1 = jnp.exp(v1 - v0)
    w0 = 1.0 / (1.0 + s1)
    topi_ref[...] = jnp.concatenate([i0, i1], axis=1)
    topw_ref[...] = jnp.concatenate([w0, 1.0 - w0], axis=1)


def _router(xf, Wg, bias_buf):
    blk = 1024
    return pl.pallas_call(
        _router_body,
        grid=(N // blk,),
        in_specs=[
            pl.BlockSpec((blk, D), lambda i: (i, 0)),
            pl.BlockSpec((E, D), lambda i: (0, 0)),
            pl.BlockSpec((1, E), lambda i: (0, 0)),
        ],
        out_specs=[
            pl.BlockSpec((blk, TOPK), lambda i: (i, 0)),
            pl.BlockSpec((blk, TOPK), lambda i: (i, 0)),
            pl.BlockSpec((blk, _DW), lambda i: (i, 0)),
        ],
        out_shape=[
            jax.ShapeDtypeStruct((N, TOPK), jnp.int32),
            jax.ShapeDtypeStruct((N, TOPK), jnp.float32),
            jax.ShapeDtypeStruct((N, _DW), jnp.int32),
        ],
    )(xf, Wg, bias_buf.reshape(1, E))


def _shared_body(x_ref, w1_ref, w2_ref, b_ref, o_ref):
    xb = _unpack_bf16(x_ref[...])
    h = lax.dot_general(xb, w1_ref[...], (((1,), (1,)), ((), ())),
                        preferred_element_type=jnp.float32)
    h = _gelu(h).astype(jnp.bfloat16)
    o = lax.dot_general(h, w2_ref[...], (((1,), (1,)), ((), ())),
                        preferred_element_type=jnp.float32)
    o_ref[...] = o + b_ref[...]


def _shared_mlp(xb, Ws1, Ws2, bs2):
    blk = 512
    return pl.pallas_call(
        _shared_body,
        grid=(N // blk,),
        in_specs=[
            pl.BlockSpec((blk, _DW), lambda i: (i, 0)),
            pl.BlockSpec((H, D), lambda i: (0, 0)),
            pl.BlockSpec((D, H), lambda i: (0, 0)),
            pl.BlockSpec((1, D), lambda i: (0, 0)),
        ],
        out_specs=pl.BlockSpec((blk, D), lambda i: (i, 0)),
        out_shape=jax.ShapeDtypeStruct((N, D), jnp.float32),
    )(xb, Ws1.astype(jnp.bfloat16), Ws2.astype(jnp.bfloat16),
      bs2.reshape(1, D))


def _meta_body(topi_hbm, st_hbm, inv_hbm, be_hbm, topi_v, st_v, inv_v, be_v):
    c = lax.axis_index("c")
    s = lax.axis_index("s")

    @pl.when(jnp.logical_and(c == 0, s == 0))
    def _():
        pltpu.sync_copy(topi_hbm, topi_v)

        zi = jnp.zeros((16,), jnp.int32)

        @pl.loop(0, G, step=16)
        def _(i):
            st_v[pl.ds(i, 16)] = zi

        def cbody(i, cnts):
            chunk = topi_v[pl.ds(i * 16, 16)]
            return tuple(cnts[e] + jnp.sum((chunk == e).astype(jnp.int32))
                         for e in range(E))

        cnt = lax.fori_loop(0, NA // 16, cbody,
                            tuple(jnp.int32(0) for _ in range(E)))

        pstart = []
        acc = jnp.int32(0)
        for e in range(E):
            pstart.append(acc)
            acc = acc + ((cnt[e] + (T - 1)) // T) * T
        bstart = [p // T for p in pstart]

        @pl.loop(0, NBPAD, step=16)
        def _(j):
            bvec = lax.iota(jnp.int32, 16) + j
            accv = jnp.full((16,), -1, jnp.int32)
            for e in range(E):
                accv = accv + (bvec >= bstart[e]).astype(jnp.int32)
            be_v[pl.ds(j, 16)] = jnp.minimum(accv, E - 1)

        def pbody(i, off):
            base = i * 16
            chunk = topi_v[pl.ds(base, 16)]
            avec = lax.iota(jnp.int32, 16) + base
            tok = avec >> 1
            inv_idx = (avec & 1) * N + (avec >> 1)
            inv_acc = jnp.zeros((16,), jnp.int32)
            newoff = []
            for e in range(E):
                m = chunk == e
                mi = m.astype(jnp.int32)
                r = jnp.cumsum(mi) - 1
                dest = off[e] + r
                plsc.store_scatter(st_v, [dest], tok, mask=m)
                inv_acc = jnp.where(m, dest, inv_acc)
                newoff.append(off[e] + jnp.sum(mi))
            plsc.store_scatter(inv_v, [inv_idx], inv_acc)
            return tuple(newoff)

        lax.fori_loop(0, NA // 16, pbody, tuple(pstart))

        pltpu.sync_copy(st_v, st_hbm)
        pltpu.sync_copy(inv_v, inv_hbm)
        pltpu.sync_copy(be_v, be_hbm)


def _meta(topi_flat):
    return pl.kernel(
        _meta_body,
        out_type=(
            jax.ShapeDtypeStruct((G,), jnp.int32),
            jax.ShapeDtypeStruct((NA,), jnp.int32),
            jax.ShapeDtypeStruct((NBPAD,), jnp.int32),
        ),
        mesh=_sc_mesh(),
        compiler_params=_SC_PARAMS,
        scratch_types=[
            pltpu.VMEM((NA,), jnp.int32),
            pltpu.VMEM((G,), jnp.int32),
            pltpu.VMEM((NA,), jnp.int32),
            pltpu.VMEM((NBPAD,), jnp.int32),
        ],
    )(topi_flat)


_RPW = G // 32
_NBUF = 4
_NWIN = _RPW // GW


def _gather_body(x_hbm, st_hbm, xs_hbm, idx_v, *bufs_sems):
    bufs = bufs_sems[:_NBUF]
    sems = bufs_sems[_NBUF:]
    c = lax.axis_index("c")
    s = lax.axis_index("s")
    wid = s * 2 + c
    base = wid * _RPW
    pltpu.sync_copy(st_hbm.at[pl.ds(base, _RPW)], idx_v)

    def fire(w, b):
        pltpu.async_copy(x_hbm.at[idx_v.at[pl.ds(w * GW, GW)]],
                         bufs[b], sems[b])

    def drain(w, b):
        pltpu.make_async_copy(x_hbm.at[idx_v.at[pl.ds(w * GW, GW)]],
                              bufs[b], sems[b]).wait()

    for k in range(min(_NBUF, _NWIN)):
        fire(k, k)
    for w in range(_NWIN):
        b = w % _NBUF
        drain(w, b)
        pltpu.sync_copy(bufs[b], xs_hbm.at[pl.ds(base + w * GW, GW)])
        if w + _NBUF < _NWIN:
            fire(w + _NBUF, b)


def _gather(xb32, st):
    return pl.kernel(
        _gather_body,
        out_type=jax.ShapeDtypeStruct((G, _DW), jnp.int32),
        mesh=_sc_mesh(),
        compiler_params=_SC_PARAMS,
        scratch_types=[pltpu.VMEM((_RPW,), jnp.int32)]
        + [pltpu.VMEM((GW, _DW), jnp.int32) for _ in range(_NBUF)]
        + [pltpu.SemaphoreType.DMA for _ in range(_NBUF)],
    )(xb32, st)


def _grouped_body(be_ref, xs_ref, w1_ref, w2_ref, b2_ref, o_ref):
    xb = _unpack_bf16(xs_ref[...])
    h = lax.dot_general(xb, w1_ref[0], (((1,), (1,)), ((), ())),
                        preferred_element_type=jnp.float32)
    h = _gelu(h).astype(jnp.bfloat16)
    o = lax.dot_general(h, w2_ref[0], (((1,), (1,)), ((), ())),
                        preferred_element_type=jnp.float32)
    o_ref[...] = o + b2_ref[0]


def _grouped(be, xs, W1, W2, b2):
    grid_spec = pltpu.PrefetchScalarGridSpec(
        num_scalar_prefetch=1,
        grid=(NB,),
        in_specs=[
            pl.BlockSpec((T, _DW), lambda i, be: (i, 0)),
            pl.BlockSpec((1, H, D), lambda i, be: (be[i], 0, 0)),
            pl.BlockSpec((1, D, H), lambda i, be: (be[i], 0, 0)),
            pl.BlockSpec((1, 1, D), lambda i, be: (be[i], 0, 0)),
        ],
        out_specs=pl.BlockSpec((T, D), lambda i, be: (i, 0)),
    )
    return pl.pallas_call(
        _grouped_body,
        grid_spec=grid_spec,
        out_shape=jax.ShapeDtypeStruct((G, D), jnp.float32),
    )(be, xs, W1, W2, b2.reshape(E, 1, D))


_TPW = N // 32


def _combine_body(sh_hbm, os_hbm, inv_hbm, w_hbm, out_hbm,
                  i0_v, i1_v, w0_v, w1_v, sh_v, t0, t1, o_v,
                  sem0, sem1, sem2):
    c = lax.axis_index("c")
    s = lax.axis_index("s")
    wid = s * 2 + c
    base = wid * _TPW
    pltpu.sync_copy(inv_hbm.at[0, pl.ds(base, _TPW)], i0_v)
    pltpu.sync_copy(inv_hbm.at[1, pl.ds(base, _TPW)], i1_v)
    pltpu.sync_copy(w_hbm.at[0, pl.ds(base, _TPW)], w0_v)
    pltpu.sync_copy(w_hbm.at[1, pl.ds(base, _TPW)], w1_v)

    @pl.loop(0, _TPW, step=CW)
    def _(j):
        pltpu.async_copy(os_hbm.at[i0_v.at[pl.ds(j, CW)]], t0, sem0)
        pltpu.async_copy(os_hbm.at[i1_v.at[pl.ds(j, CW)]], t1, sem1)
        pltpu.async_copy(sh_hbm.at[pl.ds(base + j, CW)], sh_v, sem2)
        pltpu.make_async_copy(os_hbm.at[i0_v.at[pl.ds(j, CW)]], t0, sem0).wait()
        pltpu.make_async_copy(os_hbm.at[i1_v.at[pl.ds(j, CW)]], t1, sem1).wait()
        pltpu.make_async_copy(sh_hbm.at[pl.ds(base + j, CW)], sh_v, sem2).wait()

        @pl.loop(0, CW)
        def _(r):
            ri = jnp.full((16,), j + r, jnp.int32)
            w0 = plsc.load_gather(w0_v, [ri])
            w1 = plsc.load_gather(w1_v, [ri])

            @pl.loop(0, D, step=16)
            def _(cc):
                o_v[r, pl.ds(cc, 16)] = (sh_v[r, pl.ds(cc, 16)]
                                         + w0 * t0[r, pl.ds(cc, 16)]
                                         + w1 * t1[r, pl.ds(cc, 16)])

        pltpu.sync_copy(o_v, out_hbm.at[pl.ds(base + j, CW)])


def _combine(shared, os, inv, w01):
    return pl.kernel(
        _combine_body,
        out_type=jax.ShapeDtypeStruct((N, D), jnp.float32),
        mesh=_sc_mesh(),
        compiler_params=_SC_PARAMS,
        scratch_types=[
            pltpu.VMEM((_TPW,), jnp.int32),
            pltpu.VMEM((_TPW,), jnp.int32),
            pltpu.VMEM((_TPW,), jnp.float32),
            pltpu.VMEM((_TPW,), jnp.float32),
            pltpu.VMEM((CW, D), jnp.float32),
            pltpu.VMEM((CW, D), jnp.float32),
            pltpu.VMEM((CW, D), jnp.float32),
            pltpu.VMEM((CW, D), jnp.float32),
            pltpu.SemaphoreType.DMA,
            pltpu.SemaphoreType.DMA,
            pltpu.SemaphoreType.DMA,
        ],
    )(shared, os, inv.reshape(2, N), w01)


def kernel(x, Wg, bias_buf, Ws1, Ws2, bs2, W1, W2, b2):
    xf = x.reshape(N, D)
    topi, topw, xb = _router(xf, Wg, bias_buf)
    shared = _shared_mlp(xb, Ws1, Ws2, bs2)
    st, inv, be = _meta(topi.reshape(NA))
    xs32 = _gather(xb, st)
    os = _grouped(be, xs32, W1.astype(jnp.bfloat16), W2.astype(jnp.bfloat16),
                  b2)
    w01 = topw.T.reshape(2, N)
    outf = _combine(shared, os, inv, w01)
    return outf.reshape(B, S, D)


# speedup vs baseline: 2.0119x; 1.2317x over previous
import functools

import jax
import jax.numpy as jnp
from jax import lax
from jax.experimental import pallas as pl
from jax.experimental.pallas import tpu as pltpu
from jax.experimental.pallas import tpu_sc as plsc

B, S, D = 2, 2048, 1024
H = 2048
E = 8
TOPK = 2
N = B * S
NA = N * TOPK
T = 256
NB = NA // T + E
NBPAD = 48
G = NB * T
GW = 32
CW = 16

@functools.cache
def _sc_mesh():
    return plsc.VectorSubcoreMesh(core_axis_name="c", subcore_axis_name="s")


_SC_PARAMS = pltpu.CompilerParams(needs_layout_passes=False)


def _gelu(x):
    return 0.5 * x * (1.0 + lax.erf(x * (2.0 ** -0.5)))


_DW = D // 2


def _pack_bf16(x):
    rlo = x[:, :_DW].astype(jnp.bfloat16).astype(jnp.float32)
    rhi = x[:, _DW:].astype(jnp.bfloat16).astype(jnp.float32)
    blo = lax.bitcast_convert_type(rlo, jnp.uint32) >> 16
    bhi = lax.bitcast_convert_type(rhi, jnp.uint32) & jnp.uint32(0xFFFF0000)
    return lax.bitcast_convert_type(blo | bhi, jnp.int32)


def _unpack_bf16(w32):
    u = lax.bitcast_convert_type(w32, jnp.uint32)
    lo = lax.bitcast_convert_type(u << 16, jnp.float32)
    hi = lax.bitcast_convert_type(u & jnp.uint32(0xFFFF0000), jnp.float32)
    return jnp.concatenate([lo, hi], axis=1)


def _router_body(x_ref, wg_ref, b_ref, topi_ref, topw_ref, xb_ref):
    x = x_ref[...]
    xb_ref[...] = _pack_bf16(x)
    logits = lax.dot_general(x, wg_ref[...], (((1,), (1,)), ((), ())),
                             preferred_element_type=jnp.float32)
    logits = logits + b_ref[...]
    iota = lax.broadcasted_iota(jnp.int32, logits.shape, 1)
    v0 = jnp.max(logits, axis=1, keepdims=True)
    i0 = jnp.min(jnp.where(logits == v0, iota, E), axis=1, keepdims=True)
    l2 = jnp.where(iota == i0, -jnp.inf, logits)
    v1 = jnp.max(l2, axis=1, keepdims=True)
    i1 = jnp.min(jnp.where(l2 == v1, iota, E), axis=1, keepdims=True)
    s1 = jnp.exp(v1 - v0)
    w0 = 1.0 / (1.0 + s1)
    topi_ref[...] = jnp.concatenate([i0, i1], axis=1)
    topw_ref[...] = jnp.concatenate([w0, 1.0 - w0], axis=1)


def _router(xf, Wg, bias_buf):
    blk = 1024
    return pl.pallas_call(
        _router_body,
        grid=(N // blk,),
        in_specs=[
            pl.BlockSpec((blk, D), lambda i: (i, 0)),
            pl.BlockSpec((E, D), lambda i: (0, 0)),
            pl.BlockSpec((1, E), lambda i: (0, 0)),
        ],
        out_specs=[
            pl.BlockSpec((blk, TOPK), lambda i: (i, 0)),
            pl.BlockSpec((blk, TOPK), lambda i: (i, 0)),
            pl.BlockSpec((blk, _DW), lambda i: (i, 0)),
        ],
        out_shape=[
            jax.ShapeDtypeStruct((N, TOPK), jnp.int32),
            jax.ShapeDtypeStruct((N, TOPK), jnp.float32),
            jax.ShapeDtypeStruct((N, _DW), jnp.int32),
        ],
    )(xf, Wg, bias_buf.reshape(1, E))


def _shared_body(x_ref, w1_ref, w2_ref, b_ref, o_ref):
    xb = _unpack_bf16(x_ref[...])
    h = lax.dot_general(xb, w1_ref[...], (((1,), (1,)), ((), ())),
                        preferred_element_type=jnp.float32)
    h = _gelu(h)
    o = lax.dot_general(h, w2_ref[...], (((1,), (1,)), ((), ())),
                        preferred_element_type=jnp.float32)
    o_ref[...] = o + b_ref[...]


def _shared_mlp(xb, Ws1, Ws2, bs2):
    blk = 512
    return pl.pallas_call(
        _shared_body,
        grid=(N // blk,),
        in_specs=[
            pl.BlockSpec((blk, _DW), lambda i: (i, 0)),
            pl.BlockSpec((H, D), lambda i: (0, 0)),
            pl.BlockSpec((D, H), lambda i: (0, 0)),
            pl.BlockSpec((1, D), lambda i: (0, 0)),
        ],
        out_specs=pl.BlockSpec((blk, D), lambda i: (i, 0)),
        out_shape=jax.ShapeDtypeStruct((N, D), jnp.float32),
    )(xb, Ws1, Ws2, bs2.reshape(1, D))


def _meta_body(topi_hbm, st_hbm, inv_hbm, be_hbm, topi_v, st_v, inv_v, be_v):
    c = lax.axis_index("c")
    s = lax.axis_index("s")

    @pl.when(jnp.logical_and(c == 0, s == 0))
    def _():
        pltpu.sync_copy(topi_hbm, topi_v)

        zi = jnp.zeros((16,), jnp.int32)

        @pl.loop(0, G, step=16)
        def _(i):
            st_v[pl.ds(i, 16)] = zi

        def cbody(i, cnts):
            chunk = topi_v[pl.ds(i * 16, 16)]
            return tuple(cnts[e] + jnp.sum((chunk == e).astype(jnp.int32))
                         for e in range(E))

        cnt = lax.fori_loop(0, NA // 16, cbody,
                            tuple(jnp.int32(0) for _ in range(E)))

        pstart = []
        acc = jnp.int32(0)
        for e in range(E):
            pstart.append(acc)
            acc = acc + ((cnt[e] + (T - 1)) // T) * T
        bstart = [p // T for p in pstart]

        @pl.loop(0, NBPAD, step=16)
        def _(j):
            bvec = lax.iota(jnp.int32, 16) + j
            accv = jnp.full((16,), -1, jnp.int32)
            for e in range(E):
                accv = accv + (bvec >= bstart[e]).astype(jnp.int32)
            be_v[pl.ds(j, 16)] = jnp.minimum(accv, E - 1)

        def pbody(i, off):
            base = i * 16
            chunk = topi_v[pl.ds(base, 16)]
            avec = lax.iota(jnp.int32, 16) + base
            tok = avec >> 1
            inv_idx = (avec & 1) * N + (avec >> 1)
            inv_acc = jnp.zeros((16,), jnp.int32)
            newoff = []
            for e in range(E):
                m = chunk == e
                mi = m.astype(jnp.int32)
                r = jnp.cumsum(mi) - 1
                dest = off[e] + r
                plsc.store_scatter(st_v, [dest], tok, mask=m)
                inv_acc = jnp.where(m, dest, inv_acc)
                newoff.append(off[e] + jnp.sum(mi))
            plsc.store_scatter(inv_v, [inv_idx], inv_acc)
            return tuple(newoff)

        lax.fori_loop(0, NA // 16, pbody, tuple(pstart))

        pltpu.sync_copy(st_v, st_hbm)
        pltpu.sync_copy(inv_v, inv_hbm)
        pltpu.sync_copy(be_v, be_hbm)


def _meta(topi_flat):
    return pl.kernel(
        _meta_body,
        out_type=(
            jax.ShapeDtypeStruct((G,), jnp.int32),
            jax.ShapeDtypeStruct((NA,), jnp.int32),
            jax.ShapeDtypeStruct((NBPAD,), jnp.int32),
        ),
        mesh=_sc_mesh(),
        compiler_params=_SC_PARAMS,
        scratch_types=[
            pltpu.VMEM((NA,), jnp.int32),
            pltpu.VMEM((G,), jnp.int32),
            pltpu.VMEM((NA,), jnp.int32),
            pltpu.VMEM((NBPAD,), jnp.int32),
        ],
    )(topi_flat)


_RPW = G // 32
_NBUF = 4
_NWIN = _RPW // GW


def _gather_body(x_hbm, st_hbm, xs_hbm, idx_v, *bufs_sems):
    bufs = bufs_sems[:_NBUF]
    sems = bufs_sems[_NBUF:]
    c = lax.axis_index("c")
    s = lax.axis_index("s")
    wid = s * 2 + c
    base = wid * _RPW
    pltpu.sync_copy(st_hbm.at[pl.ds(base, _RPW)], idx_v)

    def fire(w, b):
        pltpu.async_copy(x_hbm.at[idx_v.at[pl.ds(w * GW, GW)]],
                         bufs[b], sems[b])

    def drain(w, b):
        pltpu.make_async_copy(x_hbm.at[idx_v.at[pl.ds(w * GW, GW)]],
                              bufs[b], sems[b]).wait()

    for k in range(min(_NBUF, _NWIN)):
        fire(k, k)
    for w in range(_NWIN):
        b = w % _NBUF
        drain(w, b)
        pltpu.sync_copy(bufs[b], xs_hbm.at[pl.ds(base + w * GW, GW)])
        if w + _NBUF < _NWIN:
            fire(w + _NBUF, b)


def _gather(xb32, st):
    return pl.kernel(
        _gather_body,
        out_type=jax.ShapeDtypeStruct((G, _DW), jnp.int32),
        mesh=_sc_mesh(),
        compiler_params=_SC_PARAMS,
        scratch_types=[pltpu.VMEM((_RPW,), jnp.int32)]
        + [pltpu.VMEM((GW, _DW), jnp.int32) for _ in range(_NBUF)]
        + [pltpu.SemaphoreType.DMA for _ in range(_NBUF)],
    )(xb32, st)


def _grouped_body(be_ref, xs_ref, w1_ref, w2_ref, b2_ref, o_ref):
    xb = _unpack_bf16(xs_ref[...])
    h = lax.dot_general(xb, w1_ref[0], (((1,), (1,)), ((), ())),
                        preferred_element_type=jnp.float32)
    h = _gelu(h)
    o = lax.dot_general(h, w2_ref[0], (((1,), (1,)), ((), ())),
                        preferred_element_type=jnp.float32)
    o_ref[...] = o + b2_ref[0]


def _grouped(be, xs, W1, W2, b2):
    grid_spec = pltpu.PrefetchScalarGridSpec(
        num_scalar_prefetch=1,
        grid=(NB,),
        in_specs=[
            pl.BlockSpec((T, _DW), lambda i, be: (i, 0)),
            pl.BlockSpec((1, H, D), lambda i, be: (be[i], 0, 0)),
            pl.BlockSpec((1, D, H), lambda i, be: (be[i], 0, 0)),
            pl.BlockSpec((1, 1, D), lambda i, be: (be[i], 0, 0)),
        ],
        out_specs=pl.BlockSpec((T, D), lambda i, be: (i, 0)),
    )
    return pl.pallas_call(
        _grouped_body,
        grid_spec=grid_spec,
        out_shape=jax.ShapeDtypeStruct((G, D), jnp.float32),
    )(be, xs, W1, W2, b2.reshape(E, 1, D))


_TPW = N // 32


def _combine_body(sh_hbm, os_hbm, inv_hbm, w_hbm, out_hbm,
                  i0_v, i1_v, w0_v, w1_v, sh_v, t0, t1, o_v,
                  sem0, sem1, sem2):
    c = lax.axis_index("c")
    s = lax.axis_index("s")
    wid = s * 2 + c
    base = wid * _TPW
    pltpu.sync_copy(inv_hbm.at[0, pl.ds(base, _TPW)], i0_v)
    pltpu.sync_copy(inv_hbm.at[1, pl.ds(base, _TPW)], i1_v)
    pltpu.sync_copy(w_hbm.at[0, pl.ds(base, _TPW)], w0_v)
    pltpu.sync_copy(w_hbm.at[1, pl.ds(base, _TPW)], w1_v)

    @pl.loop(0, _TPW, step=CW)
    def _(j):
        pltpu.async_copy(os_hbm.at[i0_v.at[pl.ds(j, CW)]], t0, sem0)
        pltpu.async_copy(os_hbm.at[i1_v.at[pl.ds(j, CW)]], t1, sem1)
        pltpu.async_copy(sh_hbm.at[pl.ds(base + j, CW)], sh_v, sem2)
        pltpu.make_async_copy(os_hbm.at[i0_v.at[pl.ds(j, CW)]], t0, sem0).wait()
        pltpu.make_async_copy(os_hbm.at[i1_v.at[pl.ds(j, CW)]], t1, sem1).wait()
        pltpu.make_async_copy(sh_hbm.at[pl.ds(base + j, CW)], sh_v, sem2).wait()

        @pl.loop(0, CW)
        def _(r):
            ri = jnp.full((16,), j + r, jnp.int32)
            w0 = plsc.load_gather(w0_v, [ri])
            w1 = plsc.load_gather(w1_v, [ri])

            @pl.loop(0, D, step=16)
            def _(cc):
                o_v[r, pl.ds(cc, 16)] = (sh_v[r, pl.ds(cc, 16)]
                                         + w0 * t0[r, pl.ds(cc, 16)]
                                         + w1 * t1[r, pl.ds(cc, 16)])

        pltpu.sync_copy(o_v, out_hbm.at[pl.ds(base + j, CW)])


def _combine(shared, os, inv, w01):
    return pl.kernel(
        _combine_body,
        out_type=jax.ShapeDtypeStruct((N, D), jnp.float32),
        mesh=_sc_mesh(),
        compiler_params=_SC_PARAMS,
        scratch_types=[
            pltpu.VMEM((_TPW,), jnp.int32),
            pltpu.VMEM((_TPW,), jnp.int32),
            pltpu.VMEM((_TPW,), jnp.float32),
            pltpu.VMEM((_TPW,), jnp.float32),
            pltpu.VMEM((CW, D), jnp.float32),
            pltpu.VMEM((CW, D), jnp.float32),
            pltpu.VMEM((CW, D), jnp.float32),
            pltpu.VMEM((CW, D), jnp.float32),
            pltpu.SemaphoreType.DMA,
            pltpu.SemaphoreType.DMA,
            pltpu.SemaphoreType.DMA,
        ],
    )(shared, os, inv.reshape(2, N), w01)


def kernel(x, Wg, bias_buf, Ws1, Ws2, bs2, W1, W2, b2):
    xf = x.reshape(N, D)
    topi, topw, xb = _router(xf, Wg, bias_buf)
    shared = _shared_mlp(xb, Ws1, Ws2, bs2)
    st, inv, be = _meta(topi.reshape(NA))
    xs32 = _gather(xb, st)
    os = _grouped(be, xs32, W1, W2, b2)
    w01 = topw.T.reshape(2, N)
    outf = _combine(shared, os, inv, w01)
    return outf.reshape(B, S, D)


# speedup vs baseline: 2.1105x; 1.0490x over previous
import functools

import jax
import jax.numpy as jnp
from jax import lax
from jax.experimental import pallas as pl
from jax.experimental.pallas import tpu as pltpu
from jax.experimental.pallas import tpu_sc as plsc

B, S, D = 2, 2048, 1024
H = 2048
E = 8
TOPK = 2
N = B * S
NA = N * TOPK
T = 256
NB = NA // T + E
NBPAD = 48
G = NB * T
GW = 16
CW = 16

@functools.cache
def _sc_mesh():
    return plsc.VectorSubcoreMesh(core_axis_name="c", subcore_axis_name="s")


_SC_PARAMS = pltpu.CompilerParams(needs_layout_passes=False)


def _gelu(x):
    return 0.5 * x * (1.0 + lax.erf(x * (2.0 ** -0.5)))


_DW = D // 2


def _pack_bf16(x):
    rlo = x[:, :_DW].astype(jnp.bfloat16).astype(jnp.float32)
    rhi = x[:, _DW:].astype(jnp.bfloat16).astype(jnp.float32)
    blo = lax.bitcast_convert_type(rlo, jnp.uint32) >> 16
    bhi = lax.bitcast_convert_type(rhi, jnp.uint32) & jnp.uint32(0xFFFF0000)
    return lax.bitcast_convert_type(blo | bhi, jnp.int32)


def _unpack_bf16(w32):
    u = lax.bitcast_convert_type(w32, jnp.uint32)
    lo = lax.bitcast_convert_type(u << 16, jnp.float32)
    hi = lax.bitcast_convert_type(u & jnp.uint32(0xFFFF0000), jnp.float32)
    return jnp.concatenate([lo, hi], axis=1)


def _router_body(x_ref, wg_ref, b_ref, topi_ref, topw_ref, xb_ref):
    x = x_ref[...]
    xb_ref[...] = _pack_bf16(x)
    logits = lax.dot_general(x, wg_ref[...], (((1,), (1,)), ((), ())),
                             preferred_element_type=jnp.float32)
    logits = logits + b_ref[...]
    iota = lax.broadcasted_iota(jnp.int32, logits.shape, 1)
    v0 = jnp.max(logits, axis=1, keepdims=True)
    i0 = jnp.min(jnp.where(logits == v0, iota, E), axis=1, keepdims=True)
    l2 = jnp.where(iota == i0, -jnp.inf, logits)
    v1 = jnp.max(l2, axis=1, keepdims=True)
    i1 = jnp.min(jnp.where(l2 == v1, iota, E), axis=1, keepdims=True)
    s1 = jnp.exp(v1 - v0)
    w0 = 1.0 / (1.0 + s1)
    topi_ref[...] = jnp.concatenate([i0, i1], axis=1)
    topw_ref[...] = jnp.concatenate([w0, 1.0 - w0], axis=1)


def _router(xf, Wg, bias_buf):
    blk = 1024
    return pl.pallas_call(
        _router_body,
        grid=(N // blk,),
        in_specs=[
            pl.BlockSpec((blk, D), lambda i: (i, 0)),
            pl.BlockSpec((E, D), lambda i: (0, 0)),
            pl.BlockSpec((1, E), lambda i: (0, 0)),
        ],
        out_specs=[
            pl.BlockSpec((blk, TOPK), lambda i: (i, 0)),
            pl.BlockSpec((blk, TOPK), lambda i: (i, 0)),
            pl.BlockSpec((blk, _DW), lambda i: (i, 0)),
        ],
        out_shape=[
            jax.ShapeDtypeStruct((N, TOPK), jnp.int32),
            jax.ShapeDtypeStruct((N, TOPK), jnp.float32),
            jax.ShapeDtypeStruct((N, _DW), jnp.int32),
        ],
    )(xf, Wg, bias_buf.reshape(1, E))


def _shared_body(x_ref, w1_ref, w2_ref, b_ref, o_ref):
    xb = _unpack_bf16(x_ref[...])
    h = lax.dot_general(xb, w1_ref[...], (((1,), (1,)), ((), ())),
                        preferred_element_type=jnp.float32)
    h = _gelu(h)
    o = lax.dot_general(h, w2_ref[...], (((1,), (1,)), ((), ())),
                        preferred_element_type=jnp.float32)
    o_ref[...] = o + b_ref[...]


def _shared_mlp(xb, Ws1, Ws2, bs2):
    blk = 512
    return pl.pallas_call(
        _shared_body,
        grid=(N // blk,),
        in_specs=[
            pl.BlockSpec((blk, _DW), lambda i: (i, 0)),
            pl.BlockSpec((H, D), lambda i: (0, 0)),
            pl.BlockSpec((D, H), lambda i: (0, 0)),
            pl.BlockSpec((1, D), lambda i: (0, 0)),
        ],
        out_specs=pl.BlockSpec((blk, D), lambda i: (i, 0)),
        out_shape=jax.ShapeDtypeStruct((N, D), jnp.float32),
    )(xb, Ws1, Ws2, bs2.reshape(1, D))


def _meta_body(topi_hbm, st_hbm, inv_hbm, be_hbm, topi_v, st_v, inv_v, be_v):
    c = lax.axis_index("c")
    s = lax.axis_index("s")

    @pl.when(jnp.logical_and(c == 0, s == 0))
    def _():
        pltpu.sync_copy(topi_hbm, topi_v)

        zi = jnp.zeros((16,), jnp.int32)

        @pl.loop(0, G, step=16)
        def _(i):
            st_v[pl.ds(i, 16)] = zi

        def cbody(i, cnts):
            chunk = topi_v[pl.ds(i * 16, 16)]
            return tuple(cnts[e] + jnp.sum((chunk == e).astype(jnp.int32))
                         for e in range(E))

        cnt = lax.fori_loop(0, NA // 16, cbody,
                            tuple(jnp.int32(0) for _ in range(E)))

        pstart = []
        acc = jnp.int32(0)
        for e in range(E):
            pstart.append(acc)
            acc = acc + ((cnt[e] + (T - 1)) // T) * T
        bstart = [p // T for p in pstart]

        @pl.loop(0, NBPAD, step=16)
        def _(j):
            bvec = lax.iota(jnp.int32, 16) + j
            accv = jnp.full((16,), -1, jnp.int32)
            for e in range(E):
                accv = accv + (bvec >= bstart[e]).astype(jnp.int32)
            be_v[pl.ds(j, 16)] = jnp.minimum(accv, E - 1)

        def pbody(i, off):
            base = i * 16
            chunk = topi_v[pl.ds(base, 16)]
            avec = lax.iota(jnp.int32, 16) + base
            tok = avec >> 1
            inv_idx = (avec & 1) * N + (avec >> 1)
            inv_acc = jnp.zeros((16,), jnp.int32)
            newoff = []
            for e in range(E):
                m = chunk == e
                mi = m.astype(jnp.int32)
                r = jnp.cumsum(mi) - 1
                dest = off[e] + r
                plsc.store_scatter(st_v, [dest], tok, mask=m)
                inv_acc = jnp.where(m, dest, inv_acc)
                newoff.append(off[e] + jnp.sum(mi))
            plsc.store_scatter(inv_v, [inv_idx], inv_acc)
            return tuple(newoff)

        lax.fori_loop(0, NA // 16, pbody, tuple(pstart))

        pltpu.sync_copy(st_v, st_hbm)
        pltpu.sync_copy(inv_v, inv_hbm)
        pltpu.sync_copy(be_v, be_hbm)


def _meta(topi_flat):
    return pl.kernel(
        _meta_body,
        out_type=(
            jax.ShapeDtypeStruct((G,), jnp.int32),
            jax.ShapeDtypeStruct((NA,), jnp.int32),
            jax.ShapeDtypeStruct((NBPAD,), jnp.int32),
        ),
        mesh=_sc_mesh(),
        compiler_params=_SC_PARAMS,
        scratch_types=[
            pltpu.VMEM((NA,), jnp.int32),
            pltpu.VMEM((G,), jnp.int32),
            pltpu.VMEM((NA,), jnp.int32),
            pltpu.VMEM((NBPAD,), jnp.int32),
        ],
    )(topi_flat)


_RPW = G // 32
_NBUF = 8
_NWIN = _RPW // GW


_AHEAD = 4


def _gather_body(x_hbm, st_hbm, xs_hbm, idx_v, *rest):
    bufs = rest[:_NBUF]
    gsems = rest[_NBUF:2 * _NBUF]
    ssems = rest[2 * _NBUF:]
    c = lax.axis_index("c")
    s = lax.axis_index("s")
    wid = s * 2 + c
    base = wid * _RPW
    pltpu.sync_copy(st_hbm.at[pl.ds(base, _RPW)], idx_v)

    def fire(w):
        b = w % _NBUF
        pltpu.async_copy(x_hbm.at[idx_v.at[pl.ds(w * GW, GW)]],
                         bufs[b], gsems[b])

    def gwait(w):
        b = w % _NBUF
        pltpu.make_async_copy(x_hbm.at[idx_v.at[pl.ds(w * GW, GW)]],
                              bufs[b], gsems[b]).wait()

    def swait(w):
        b = w % _NBUF
        pltpu.make_async_copy(bufs[b], xs_hbm.at[pl.ds(base + w * GW, GW)],
                              ssems[b]).wait()

    for k in range(min(_AHEAD, _NWIN)):
        fire(k)
    for w in range(_NWIN):
        b = w % _NBUF
        gwait(w)
        pltpu.async_copy(bufs[b], xs_hbm.at[pl.ds(base + w * GW, GW)],
                         ssems[b])
        nxt = w + _AHEAD
        if nxt < _NWIN:
            if nxt >= _NBUF:
                swait(nxt - _NBUF)
            fire(nxt)
    for w in range(max(0, _NWIN - _NBUF), _NWIN):
        swait(w)


def _gather(xb32, st):
    return pl.kernel(
        _gather_body,
        out_type=jax.ShapeDtypeStruct((G, _DW), jnp.int32),
        mesh=_sc_mesh(),
        compiler_params=_SC_PARAMS,
        scratch_types=[pltpu.VMEM((_RPW,), jnp.int32)]
        + [pltpu.VMEM((GW, _DW), jnp.int32) for _ in range(_NBUF)]
        + [pltpu.SemaphoreType.DMA for _ in range(2 * _NBUF)],
    )(xb32, st)


def _grouped_body(be_ref, xs_ref, w1_ref, w2_ref, b2_ref, o_ref):
    xb = _unpack_bf16(xs_ref[...])
    h = lax.dot_general(xb, w1_ref[0], (((1,), (1,)), ((), ())),
                        preferred_element_type=jnp.float32)
    h = _gelu(h)
    o = lax.dot_general(h, w2_ref[0], (((1,), (1,)), ((), ())),
                        preferred_element_type=jnp.float32)
    o_ref[...] = o + b2_ref[0]


def _grouped(be, xs, W1, W2, b2):
    grid_spec = pltpu.PrefetchScalarGridSpec(
        num_scalar_prefetch=1,
        grid=(NB,),
        in_specs=[
            pl.BlockSpec((T, _DW), lambda i, be: (i, 0)),
            pl.BlockSpec((1, H, D), lambda i, be: (be[i], 0, 0)),
            pl.BlockSpec((1, D, H), lambda i, be: (be[i], 0, 0)),
            pl.BlockSpec((1, 1, D), lambda i, be: (be[i], 0, 0)),
        ],
        out_specs=pl.BlockSpec((T, D), lambda i, be: (i, 0)),
    )
    return pl.pallas_call(
        _grouped_body,
        grid_spec=grid_spec,
        out_shape=jax.ShapeDtypeStruct((G, D), jnp.float32),
    )(be, xs, W1, W2, b2.reshape(E, 1, D))


_TPW = N // 32


def _combine_body(sh_hbm, os_hbm, inv_hbm, w_hbm, out_hbm,
                  i0_v, i1_v, w0_v, w1_v, o_v, *sets_flat):
    sets = (sets_flat[0:6], sets_flat[6:12])
    c = lax.axis_index("c")
    s = lax.axis_index("s")
    wid = s * 2 + c
    base = wid * _TPW
    pltpu.sync_copy(inv_hbm.at[0, pl.ds(base, _TPW)], i0_v)
    pltpu.sync_copy(inv_hbm.at[1, pl.ds(base, _TPW)], i1_v)
    pltpu.sync_copy(w_hbm.at[0, pl.ds(base, _TPW)], w0_v)
    pltpu.sync_copy(w_hbm.at[1, pl.ds(base, _TPW)], w1_v)

    nwin = _TPW // CW

    def fire(jw):
        t0, t1, sh_v, sem0, sem1, sem2 = sets[jw % 2]
        j = jw * CW
        pltpu.async_copy(os_hbm.at[i0_v.at[pl.ds(j, CW)]], t0, sem0)
        pltpu.async_copy(os_hbm.at[i1_v.at[pl.ds(j, CW)]], t1, sem1)
        pltpu.async_copy(sh_hbm.at[pl.ds(base + j, CW)], sh_v, sem2)

    def drain(jw):
        t0, t1, sh_v, sem0, sem1, sem2 = sets[jw % 2]
        j = jw * CW
        pltpu.make_async_copy(os_hbm.at[i0_v.at[pl.ds(j, CW)]], t0,
                              sem0).wait()
        pltpu.make_async_copy(os_hbm.at[i1_v.at[pl.ds(j, CW)]], t1,
                              sem1).wait()
        pltpu.make_async_copy(sh_hbm.at[pl.ds(base + j, CW)], sh_v,
                              sem2).wait()

    fire(0)
    for jw in range(nwin):
        t0, t1, sh_v, _, _, _ = sets[jw % 2]
        if jw + 1 < nwin:
            fire(jw + 1)
        drain(jw)
        j = jw * CW

        @pl.loop(0, CW)
        def _(r):
            ri = jnp.full((16,), j + r, jnp.int32)
            w0 = plsc.load_gather(w0_v, [ri])
            w1 = plsc.load_gather(w1_v, [ri])

            @pl.loop(0, D, step=16)
            def _(cc):
                o_v[r, pl.ds(cc, 16)] = (sh_v[r, pl.ds(cc, 16)]
                                         + w0 * t0[r, pl.ds(cc, 16)]
                                         + w1 * t1[r, pl.ds(cc, 16)])

        pltpu.sync_copy(o_v, out_hbm.at[pl.ds(base + j, CW)])


def _combine(shared, os, inv, w01):
    bufset = [
        pltpu.VMEM((CW, D), jnp.float32),
        pltpu.VMEM((CW, D), jnp.float32),
        pltpu.VMEM((CW, D), jnp.float32),
        pltpu.SemaphoreType.DMA,
        pltpu.SemaphoreType.DMA,
        pltpu.SemaphoreType.DMA,
    ]
    return pl.kernel(
        _combine_body,
        out_type=jax.ShapeDtypeStruct((N, D), jnp.float32),
        mesh=_sc_mesh(),
        compiler_params=_SC_PARAMS,
        scratch_types=[
            pltpu.VMEM((_TPW,), jnp.int32),
            pltpu.VMEM((_TPW,), jnp.int32),
            pltpu.VMEM((_TPW,), jnp.float32),
            pltpu.VMEM((_TPW,), jnp.float32),
            pltpu.VMEM((CW, D), jnp.float32),
        ] + bufset + bufset,
    )(shared, os, inv.reshape(2, N), w01)


def kernel(x, Wg, bias_buf, Ws1, Ws2, bs2, W1, W2, b2):
    xf = x.reshape(N, D)
    topi, topw, xb = _router(xf, Wg, bias_buf)
    shared = _shared_mlp(xb, Ws1, Ws2, bs2)
    st, inv, be = _meta(topi.reshape(NA))
    xs32 = _gather(xb, st)
    os = _grouped(be, xs32, W1, W2, b2)
    w01 = topw.T.reshape(2, N)
    outf = _combine(shared, os, inv, w01)
    return outf.reshape(B, S, D)
